# Initial kernel scaffold; baseline (speedup 1.0000x reference)
import jax
import jax.numpy as jnp
from jax.experimental import pallas as pl


def kernel(support_xf, support_y, query_xf, query_y):
    raise NotImplementedError("write your pallas kernel here")


# speedup vs baseline: 10.5266x; 10.5266x over previous
import jax
import jax.numpy as jnp
from jax import lax
from jax.experimental import pallas as pl

_N_WAY = 5
_K_SHOT = 5
_NEIGHBOR_K = 3
_B = 8
_Q = 75
_C = 640
_HW = 25
_QR = _Q * _HW
_SR = _N_WAY * _K_SHOT * _HW
_PER_CLASS = _K_SHOT * _HW
_EPS = 1e-8
_NEG = -1e30


def _body(qv_ref, sv_ref, qy_ref, out_ref):
    b = pl.program_id(0)

    qv = qv_ref[0]
    sv = sv_ref[0]

    qn = jnp.sqrt(jnp.sum(qv * qv, axis=1, keepdims=True)) + _EPS
    qv = qv / qn
    sn = jnp.sqrt(jnp.sum(sv * sv, axis=1, keepdims=True)) + _EPS
    sv = sv / sn

    col = lax.broadcasted_iota(jnp.int32, (_QR, _PER_CLASS), 1)

    s3_cols = []
    for n in range(_N_WAY):
        svn = sv[n * _PER_CLASS:(n + 1) * _PER_CLASS, :]
        x = lax.dot_general(qv, svn, (((1,), (1,)), ((), ())),
                            preferred_element_type=jnp.float32)
        s3 = jnp.zeros((_QR, 1), jnp.float32)
        for t in range(_NEIGHBOR_K):
            m = jnp.max(x, axis=1, keepdims=True)
            s3 = s3 + m
            if t + 1 < _NEIGHBOR_K:
                hit = x >= m
                first = jnp.min(jnp.where(hit, col, _PER_CLASS), axis=1,
                                keepdims=True)
                x = jnp.where(col == first, _NEG, x)
        s3_cols.append(s3 / float(_NEIGHBOR_K))
    s3 = jnp.concatenate(s3_cols, axis=1)

    gcol = lax.broadcasted_iota(jnp.int32, (_Q, _QR), 1)
    grow = lax.broadcasted_iota(jnp.int32, (_Q, _QR), 0)
    g = jnp.where(gcol // _HW == grow, 1.0, 0.0)
    logits = lax.dot_general(g, s3, (((1,), (0,)), ((), ())),
                             preferred_element_type=jnp.float32)

    mx = jnp.max(logits, axis=1, keepdims=True)
    sh = logits - mx
    lse = jnp.log(jnp.sum(jnp.exp(sh), axis=1, keepdims=True))
    logp = sh - lse
    lab = qy_ref[0]
    onehot = (lax.broadcasted_iota(jnp.int32, (_Q, _N_WAY), 1) == lab)
    batch_loss = -jnp.sum(jnp.where(onehot, logp, 0.0), axis=(0, 1),
                          keepdims=True) / float(_B * _Q)

    @pl.when(b == 0)
    def _init():
        out_ref[:, :] = jnp.zeros((1, 1), jnp.float32)

    out_ref[:, :] += batch_loss


def kernel(support_xf, support_y, query_xf, query_y):
    del support_y
    b, q, c, h, w = query_xf.shape
    hw = h * w
    qv = query_xf.reshape(b, q, c, hw).transpose(0, 1, 3, 2).reshape(b, q * hw, c)
    sv = support_xf.reshape(b, _SR // hw, c, hw).transpose(0, 1, 3, 2)
    sv = sv.reshape(b, _SR, c)
    qy = query_y.reshape(b, q, 1).astype(jnp.int32)

    loss = pl.pallas_call(
        _body,
        grid=(b,),
        in_specs=[
            pl.BlockSpec((1, q * hw, c), lambda i: (i, 0, 0)),
            pl.BlockSpec((1, _SR, c), lambda i: (i, 0, 0)),
            pl.BlockSpec((1, q, 1), lambda i: (i, 0, 0)),
        ],
        out_specs=pl.BlockSpec((1, 1), lambda i: (0, 0)),
        out_shape=jax.ShapeDtypeStruct((1, 1), jnp.float32),
    )(qv, sv, qy)
    return loss[0, 0]
